# baseline (speedup 1.0000x reference)
import jax
import jax.numpy as jnp
from jax import lax
from jax.experimental import pallas as pl
from jax.experimental.pallas import tpu as pltpu
from jax.experimental.pallas import tpu_sc as plsc

N = 10000
E = 320000
D = 128
H = 128
C = 64

NC = 2
NS = 16
NW = NC * NS
CH = 128
K1 = 8 * -(-E // (NW * CH * 8))
EPAD = NW * K1 * CH
RPT = 8 * -(-(N + 1) // (NS * 8))
NPAD = RPT * NS
DW = 16
FA = D + DW
SUP = 8
KA = 152
KB = 2 * K1 - KA
GA = KA // SUP
GB = KB // SUP


def _sc_aggregate(feat, src_flat, dst_2d):
    F = feat.shape[1]
    NBUF = 2 if F > 96 else 4
    mesh = plsc.VectorSubcoreMesh(
        core_axis_name="c", subcore_axis_name="s", num_cores=NC, num_subcores=NS
    )
    out_type = [jax.ShapeDtypeStruct((NC, NPAD, F), jnp.float32)]
    scratch = (
        [pltpu.VMEM((SUP * CH,), jnp.int32),
         pltpu.VMEM((SUP, CH), jnp.int32)]
        + [pltpu.VMEM((CH, F), jnp.float32) for _ in range(NBUF)]
        + [pltpu.VMEM_SHARED((NPAD, F), jnp.float32)]
        + [pltpu.SemaphoreType.DMA for _ in range(2 * NBUF)]
    )
    zeros_f = jnp.zeros((NPAD, F), jnp.float32)

    def body(feat_h, src_h, dst_h, zf_h, agg_out, src_v, dst_v, *rest):
        bufs = rest[:NBUF]
        acc_sh = rest[NBUF]
        gsems = rest[NBUF + 1:2 * NBUF + 1]
        ssems = rest[2 * NBUF + 1:]
        cid = lax.axis_index("c")
        sid = lax.axis_index("s")
        base_chunk = lax.select(cid == 0, sid * KA, NS * KA + sid * KB)
        trips = lax.select(cid == 0, GA, GB)
        r0 = sid * RPT
        pltpu.sync_copy(zf_h.at[pl.ds(r0, RPT)], acc_sh.at[pl.ds(r0, RPT)])
        plsc.subcore_barrier()

        def gather(j):
            return pltpu.async_copy(
                feat_h.at[src_v.at[pl.ds(j * CH, CH)]],
                bufs[j % NBUF], gsems[j % NBUF])

        def step(g, carry):
            c0 = base_chunk + g * SUP
            pltpu.sync_copy(src_h.at[pl.ds(c0 * CH, SUP * CH)], src_v)
            pltpu.sync_copy(dst_h.at[pl.ds(c0, SUP)], dst_v)
            gd = {}
            sd = {}
            for j in range(NBUF - 1):
                gd[j] = gather(j)
            for j in range(SUP):
                b = j % NBUF
                gd[j].wait()
                sd[j] = pltpu.async_copy(
                    bufs[b], acc_sh.at[dst_v.at[j]], ssems[b], add=True)
                nxt = j + NBUF - 1
                if nxt < SUP:
                    if j > 0:
                        sd[j - 1].wait()
                    gd[nxt] = gather(nxt)
            for j in range(max(0, SUP - NBUF), SUP):
                sd[j].wait()
            return carry

        lax.fori_loop(0, trips, step, 0)
        plsc.subcore_barrier()
        pltpu.sync_copy(acc_sh.at[pl.ds(r0, RPT)],
                        agg_out.at[cid, pl.ds(r0, RPT)])

    fn = pl.kernel(
        body, out_type=out_type, mesh=mesh, scratch_types=scratch,
        compiler_params=pltpu.CompilerParams(use_tc_tiling_on_sc=False),
    )
    return fn(feat, src_flat, dst_2d, zeros_f)[0]


BR = 1000


def _tc_layer1_body(x_ref, agg_ref, ws1_ref, wn1_ref, b1_ref,
                    wn2_ref, ws2_ref, y2_ref, hs2_ref, inv_ref):
    a = agg_ref[0] + agg_ref[1]
    deg = a[:, D:D + 1]
    inv = 1.0 / jnp.maximum(deg, 1.0)
    hn = a[:, :D] * inv
    h = (jnp.dot(x_ref[...], ws1_ref[...], preferred_element_type=jnp.float32)
         + jnp.dot(hn, wn1_ref[...], preferred_element_type=jnp.float32)
         + b1_ref[...])
    h = jnp.maximum(h, 0.0)
    y2_ref[...] = jnp.dot(h, wn2_ref[...], preferred_element_type=jnp.float32)
    hs2_ref[...] = jnp.dot(h, ws2_ref[...], preferred_element_type=jnp.float32)
    inv_ref[...] = jnp.broadcast_to(inv, (BR, 8))


def _tc_layer2_body(hs2_ref, agg_ref, inv_ref, b2_ref, out_ref):
    agg = agg_ref[0] + agg_ref[1]
    out_ref[...] = hs2_ref[...] + agg * inv_ref[:, :1] + b2_ref[...]


def kernel(x, edge_index, W_self1, W_neigh1, b1, W_self2, W_neigh2, b2):
    src = edge_index[0]
    dst = edge_index[1]
    pad = EPAD - E
    src_p = jnp.concatenate([src, jnp.zeros((pad,), jnp.int32)])
    dst_p = jnp.concatenate([dst, jnp.full((pad,), N, jnp.int32)])
    dst_2d = dst_p.reshape(NW * K1, CH)
    xa = jnp.concatenate([x, jnp.ones((N, DW), jnp.float32)], axis=1)

    agg1 = _sc_aggregate(xa, src_p, dst_2d)

    grid = N // BR
    y2, hs2, inv = pl.pallas_call(
        _tc_layer1_body,
        grid=(grid,),
        in_specs=[
            pl.BlockSpec((BR, D), lambda i: (i, 0)),
            pl.BlockSpec((NC, BR, FA), lambda i: (0, i, 0)),
            pl.BlockSpec((D, H), lambda i: (0, 0)),
            pl.BlockSpec((D, H), lambda i: (0, 0)),
            pl.BlockSpec((1, H), lambda i: (0, 0)),
            pl.BlockSpec((H, C), lambda i: (0, 0)),
            pl.BlockSpec((H, C), lambda i: (0, 0)),
        ],
        out_specs=[
            pl.BlockSpec((BR, C), lambda i: (i, 0)),
            pl.BlockSpec((BR, C), lambda i: (i, 0)),
            pl.BlockSpec((BR, 8), lambda i: (i, 0)),
        ],
        out_shape=[
            jax.ShapeDtypeStruct((N, C), jnp.float32),
            jax.ShapeDtypeStruct((N, C), jnp.float32),
            jax.ShapeDtypeStruct((N, 8), jnp.float32),
        ],
    )(x, agg1, W_self1, W_neigh1, b1.reshape(1, H), W_neigh2, W_self2)

    agg2 = _sc_aggregate(y2, src_p, dst_2d)

    out = pl.pallas_call(
        _tc_layer2_body,
        grid=(grid,),
        in_specs=[
            pl.BlockSpec((BR, C), lambda i: (i, 0)),
            pl.BlockSpec((NC, BR, C), lambda i: (0, i, 0)),
            pl.BlockSpec((BR, 8), lambda i: (i, 0)),
            pl.BlockSpec((1, C), lambda i: (0, 0)),
        ],
        out_specs=pl.BlockSpec((BR, C), lambda i: (i, 0)),
        out_shape=jax.ShapeDtypeStruct((N, C), jnp.float32),
    )(hs2, agg2, inv, b2.reshape(1, C))
    return out


# speedup vs baseline: 1.0606x; 1.0606x over previous
import jax
import jax.numpy as jnp
from jax import lax
from jax.experimental import pallas as pl
from jax.experimental.pallas import tpu as pltpu
from jax.experimental.pallas import tpu_sc as plsc

N = 10000
E = 320000
D = 128
H = 128
C = 64

NC = 2
NS = 16
NW = NC * NS
CH = 128
K1 = 8 * -(-E // (NW * CH * 8))
EPAD = NW * K1 * CH
RPT = 8 * -(-(N + 1) // (NS * 8))
NPAD = RPT * NS
DW = 16
FA = D + DW
SUP = 8
KA = 144
KB = 2 * K1 - KA
GA = KA // SUP
GB = KB // SUP


def _sc_aggregate(feat, src_flat, dst_2d):
    F = feat.shape[1]
    NBUF = 2 if F > 96 else 4
    mesh = plsc.VectorSubcoreMesh(
        core_axis_name="c", subcore_axis_name="s", num_cores=NC, num_subcores=NS
    )
    out_type = [jax.ShapeDtypeStruct((NC, NPAD, F), jnp.float32)]
    scratch = (
        [pltpu.VMEM((SUP * CH,), jnp.int32),
         pltpu.VMEM((SUP, CH), jnp.int32)]
        + [pltpu.VMEM((CH, F), jnp.float32) for _ in range(NBUF)]
        + [pltpu.VMEM_SHARED((NPAD, F), jnp.float32)]
        + [pltpu.SemaphoreType.DMA for _ in range(2 * NBUF)]
    )
    zeros_f = jnp.zeros((NPAD, F), jnp.float32)

    def body(feat_h, src_h, dst_h, zf_h, agg_out, src_v, dst_v, *rest):
        bufs = rest[:NBUF]
        acc_sh = rest[NBUF]
        gsems = rest[NBUF + 1:2 * NBUF + 1]
        ssems = rest[2 * NBUF + 1:]
        cid = lax.axis_index("c")
        sid = lax.axis_index("s")
        base_chunk = lax.select(cid == 0, sid * KA, NS * KA + sid * KB)
        trips = lax.select(cid == 0, GA, GB)
        r0 = sid * RPT
        pltpu.sync_copy(zf_h.at[pl.ds(r0, RPT)], acc_sh.at[pl.ds(r0, RPT)])
        plsc.subcore_barrier()

        def gather(j):
            return pltpu.async_copy(
                feat_h.at[src_v.at[pl.ds(j * CH, CH)]],
                bufs[j % NBUF], gsems[j % NBUF])

        def step(g, carry):
            c0 = base_chunk + g * SUP
            pltpu.sync_copy(src_h.at[pl.ds(c0 * CH, SUP * CH)], src_v)
            pltpu.sync_copy(dst_h.at[pl.ds(c0, SUP)], dst_v)
            gd = {}
            sd = {}
            for j in range(NBUF - 1):
                gd[j] = gather(j)
            for j in range(SUP):
                b = j % NBUF
                gd[j].wait()
                sd[j] = pltpu.async_copy(
                    bufs[b], acc_sh.at[dst_v.at[j]], ssems[b], add=True)
                nxt = j + NBUF - 1
                if nxt < SUP:
                    if j > 0:
                        sd[j - 1].wait()
                    gd[nxt] = gather(nxt)
            for j in range(max(0, SUP - NBUF), SUP):
                sd[j].wait()
            return carry

        lax.fori_loop(0, trips, step, 0)
        plsc.subcore_barrier()
        pltpu.sync_copy(acc_sh.at[pl.ds(r0, RPT)],
                        agg_out.at[cid, pl.ds(r0, RPT)])

    fn = pl.kernel(
        body, out_type=out_type, mesh=mesh, scratch_types=scratch,
        compiler_params=pltpu.CompilerParams(use_tc_tiling_on_sc=False),
    )
    return fn(feat, src_flat, dst_2d, zeros_f)[0]


BR = 1000


def _tc_layer1_body(x_ref, agg_ref, ws1_ref, wn1_ref, b1_ref,
                    wn2_ref, ws2_ref, y2_ref, hs2_ref, inv_ref):
    a = agg_ref[0] + agg_ref[1]
    deg = a[:, D:D + 1]
    inv = 1.0 / jnp.maximum(deg, 1.0)
    hn = a[:, :D] * inv
    h = (jnp.dot(x_ref[...], ws1_ref[...], preferred_element_type=jnp.float32)
         + jnp.dot(hn, wn1_ref[...], preferred_element_type=jnp.float32)
         + b1_ref[...])
    h = jnp.maximum(h, 0.0)
    y2_ref[...] = jnp.dot(h, wn2_ref[...], preferred_element_type=jnp.float32)
    hs2_ref[...] = jnp.dot(h, ws2_ref[...], preferred_element_type=jnp.float32)
    inv_ref[...] = jnp.broadcast_to(inv, (BR, 8))


def _tc_layer2_body(hs2_ref, agg_ref, inv_ref, b2_ref, out_ref):
    agg = agg_ref[0] + agg_ref[1]
    out_ref[...] = hs2_ref[...] + agg * inv_ref[:, :1] + b2_ref[...]


def kernel(x, edge_index, W_self1, W_neigh1, b1, W_self2, W_neigh2, b2):
    src = edge_index[0]
    dst = edge_index[1]
    pad = EPAD - E
    src_p = jnp.concatenate([src, jnp.zeros((pad,), jnp.int32)])
    dst_p = jnp.concatenate([dst, jnp.full((pad,), N, jnp.int32)])
    dst_2d = dst_p.reshape(NW * K1, CH)
    xa = jnp.concatenate([x, jnp.ones((N, DW), jnp.float32)], axis=1)

    agg1 = _sc_aggregate(xa, src_p, dst_2d)

    grid = N // BR
    y2, hs2, inv = pl.pallas_call(
        _tc_layer1_body,
        grid=(grid,),
        in_specs=[
            pl.BlockSpec((BR, D), lambda i: (i, 0)),
            pl.BlockSpec((NC, BR, FA), lambda i: (0, i, 0)),
            pl.BlockSpec((D, H), lambda i: (0, 0)),
            pl.BlockSpec((D, H), lambda i: (0, 0)),
            pl.BlockSpec((1, H), lambda i: (0, 0)),
            pl.BlockSpec((H, C), lambda i: (0, 0)),
            pl.BlockSpec((H, C), lambda i: (0, 0)),
        ],
        out_specs=[
            pl.BlockSpec((BR, C), lambda i: (i, 0)),
            pl.BlockSpec((BR, C), lambda i: (i, 0)),
            pl.BlockSpec((BR, 8), lambda i: (i, 0)),
        ],
        out_shape=[
            jax.ShapeDtypeStruct((N, C), jnp.float32),
            jax.ShapeDtypeStruct((N, C), jnp.float32),
            jax.ShapeDtypeStruct((N, 8), jnp.float32),
        ],
    )(x, agg1, W_self1, W_neigh1, b1.reshape(1, H), W_neigh2, W_self2)

    agg2 = _sc_aggregate(y2, src_p, dst_2d)

    out = pl.pallas_call(
        _tc_layer2_body,
        grid=(grid,),
        in_specs=[
            pl.BlockSpec((BR, C), lambda i: (i, 0)),
            pl.BlockSpec((NC, BR, C), lambda i: (0, i, 0)),
            pl.BlockSpec((BR, 8), lambda i: (i, 0)),
            pl.BlockSpec((1, C), lambda i: (0, 0)),
        ],
        out_specs=pl.BlockSpec((BR, C), lambda i: (i, 0)),
        out_shape=jax.ShapeDtypeStruct((N, C), jnp.float32),
    )(hs2, agg2, inv, b2.reshape(1, C))
    return out


# speedup vs baseline: 1.1184x; 1.0546x over previous
import jax
import jax.numpy as jnp
from jax import lax
from jax.experimental import pallas as pl
from jax.experimental.pallas import tpu as pltpu
from jax.experimental.pallas import tpu_sc as plsc

N = 10000
E = 320000
D = 128
H = 128
C = 64

NC = 2
NS = 16
NW = NC * NS
CH = 128
K1 = 8 * -(-E // (NW * CH * 8))
EPAD = NW * K1 * CH
RPT = 8 * -(-(N + 1) // (NS * 8))
NPAD = RPT * NS
DW = 16
SUP = 8
KA = 144
KB = 2 * K1 - KA
GA = KA // SUP
GB = KB // SUP
L = 16


def _sc_aggregate(feat, src_flat, dst_2d, with_deg):
    F = feat.shape[1]
    NBUF = 2 if F > 96 else 4
    mesh = plsc.VectorSubcoreMesh(
        core_axis_name="c", subcore_axis_name="s", num_cores=NC, num_subcores=NS
    )
    out_type = [jax.ShapeDtypeStruct((NC, NPAD, F), jnp.float32)]
    if with_deg:
        out_type.append(jax.ShapeDtypeStruct((NC, NPAD, DW), jnp.float32))
    scratch = (
        [pltpu.VMEM((SUP * CH,), jnp.int32),
         pltpu.VMEM((SUP, CH), jnp.int32)]
        + [pltpu.VMEM((CH, F), jnp.float32) for _ in range(NBUF)]
        + [pltpu.VMEM_SHARED((NPAD, F), jnp.float32)]
        + [pltpu.SemaphoreType.DMA for _ in range(2 * NBUF)]
    )
    if with_deg:
        scratch += [
            pltpu.VMEM((CH, DW), jnp.float32),
            pltpu.VMEM_SHARED((NPAD, DW), jnp.float32),
            pltpu.SemaphoreType.DMA,
        ]

    def body(feat_h, src_h, dst_h, *rest):
        outs = rest[:len(out_type)]
        rest = rest[len(out_type):]
        agg_out = outs[0]
        src_v, dst_v = rest[0], rest[1]
        bufs = rest[2:2 + NBUF]
        acc_sh = rest[2 + NBUF]
        gsems = rest[3 + NBUF:3 + 2 * NBUF]
        ssems = rest[3 + 2 * NBUF:3 + 3 * NBUF]
        if with_deg:
            deg_out = outs[1]
            ones_v, deg_sh, dsem = rest[3 + 3 * NBUF:]
        cid = lax.axis_index("c")
        sid = lax.axis_index("s")
        base_chunk = lax.select(cid == 0, sid * KA, NS * KA + sid * KB)
        trips = lax.select(cid == 0, GA, GB)
        r0 = sid * RPT

        zv = jnp.zeros((L,), jnp.float32)

        def initrow(i, carry):
            for q in range(F // L):
                bufs[0][i, pl.ds(q * L, L)] = zv
            if with_deg:
                ones_v[i, :] = zv + 1.0
            return carry

        lax.fori_loop(0, CH, initrow, 0)
        nfull = RPT // CH
        rem = RPT - nfull * CH
        for k in range(nfull):
            pltpu.sync_copy(bufs[0], acc_sh.at[pl.ds(r0 + k * CH, CH)])
        if rem:
            pltpu.sync_copy(bufs[0].at[pl.ds(0, rem)],
                            acc_sh.at[pl.ds(r0 + nfull * CH, rem)])
        if with_deg:
            for k in range(nfull):
                pltpu.sync_copy(bufs[0].at[pl.ds(0, CH), pl.ds(0, DW)],
                                deg_sh.at[pl.ds(r0 + k * CH, CH)])
            if rem:
                pltpu.sync_copy(bufs[0].at[pl.ds(0, rem), pl.ds(0, DW)],
                                deg_sh.at[pl.ds(r0 + nfull * CH, rem)])
        plsc.subcore_barrier()

        def gather(j):
            return pltpu.async_copy(
                feat_h.at[src_v.at[pl.ds(j * CH, CH)]],
                bufs[j % NBUF], gsems[j % NBUF])

        def step(g, carry):
            c0 = base_chunk + g * SUP
            pltpu.sync_copy(src_h.at[pl.ds(c0 * CH, SUP * CH)], src_v)
            pltpu.sync_copy(dst_h.at[pl.ds(c0, SUP)], dst_v)
            gd = {}
            sd = {}
            dd = {}
            for j in range(NBUF - 1):
                gd[j] = gather(j)
            for j in range(SUP):
                b = j % NBUF
                gd[j].wait()
                sd[j] = pltpu.async_copy(
                    bufs[b], acc_sh.at[dst_v.at[j]], ssems[b], add=True)
                if with_deg:
                    dd[j] = pltpu.async_copy(
                        ones_v, deg_sh.at[dst_v.at[j]], dsem, add=True)
                nxt = j + NBUF - 1
                if nxt < SUP:
                    if j > 0:
                        sd[j - 1].wait()
                    gd[nxt] = gather(nxt)
            for j in range(max(0, SUP - NBUF), SUP):
                sd[j].wait()
            if with_deg:
                for j in range(SUP):
                    dd[j].wait()
            return carry

        lax.fori_loop(0, trips, step, 0)
        plsc.subcore_barrier()
        pltpu.sync_copy(acc_sh.at[pl.ds(r0, RPT)],
                        agg_out.at[cid, pl.ds(r0, RPT)])
        if with_deg:
            pltpu.sync_copy(deg_sh.at[pl.ds(r0, RPT)],
                            deg_out.at[cid, pl.ds(r0, RPT)])

    fn = pl.kernel(
        body, out_type=out_type, mesh=mesh, scratch_types=scratch,
        compiler_params=pltpu.CompilerParams(use_tc_tiling_on_sc=False),
    )
    return fn(feat, src_flat, dst_2d)


BR = 1000


def _tc_layer1_body(x_ref, agg_ref, deg_ref, ws1_ref, wn1_ref, b1_ref,
                    wn2_ref, ws2_ref, y2_ref, hs2_ref, inv_ref):
    a = agg_ref[0] + agg_ref[1]
    deg = deg_ref[0][:, :1] + deg_ref[1][:, :1]
    inv = 1.0 / jnp.maximum(deg, 1.0)
    hn = a * inv
    h = (jnp.dot(x_ref[...], ws1_ref[...], preferred_element_type=jnp.float32)
         + jnp.dot(hn, wn1_ref[...], preferred_element_type=jnp.float32)
         + b1_ref[...])
    h = jnp.maximum(h, 0.0)
    y2_ref[...] = jnp.dot(h, wn2_ref[...], preferred_element_type=jnp.float32)
    hs2_ref[...] = jnp.dot(h, ws2_ref[...], preferred_element_type=jnp.float32)
    inv_ref[...] = jnp.broadcast_to(inv, (BR, 8))


def _tc_layer2_body(hs2_ref, agg_ref, inv_ref, b2_ref, out_ref):
    agg = agg_ref[0] + agg_ref[1]
    out_ref[...] = hs2_ref[...] + agg * inv_ref[:, :1] + b2_ref[...]


def kernel(x, edge_index, W_self1, W_neigh1, b1, W_self2, W_neigh2, b2):
    src = edge_index[0]
    dst = edge_index[1]
    pad = EPAD - E
    src_p = jnp.concatenate([src, jnp.zeros((pad,), jnp.int32)])
    dst_p = jnp.concatenate([dst, jnp.full((pad,), N, jnp.int32)])
    dst_2d = dst_p.reshape(NW * K1, CH)

    agg1, deg = _sc_aggregate(x, src_p, dst_2d, with_deg=True)

    grid = N // BR
    y2, hs2, inv = pl.pallas_call(
        _tc_layer1_body,
        grid=(grid,),
        in_specs=[
            pl.BlockSpec((BR, D), lambda i: (i, 0)),
            pl.BlockSpec((NC, BR, D), lambda i: (0, i, 0)),
            pl.BlockSpec((NC, BR, DW), lambda i: (0, i, 0)),
            pl.BlockSpec((D, H), lambda i: (0, 0)),
            pl.BlockSpec((D, H), lambda i: (0, 0)),
            pl.BlockSpec((1, H), lambda i: (0, 0)),
            pl.BlockSpec((H, C), lambda i: (0, 0)),
            pl.BlockSpec((H, C), lambda i: (0, 0)),
        ],
        out_specs=[
            pl.BlockSpec((BR, C), lambda i: (i, 0)),
            pl.BlockSpec((BR, C), lambda i: (i, 0)),
            pl.BlockSpec((BR, 8), lambda i: (i, 0)),
        ],
        out_shape=[
            jax.ShapeDtypeStruct((N, C), jnp.float32),
            jax.ShapeDtypeStruct((N, C), jnp.float32),
            jax.ShapeDtypeStruct((N, 8), jnp.float32),
        ],
    )(x, agg1, deg, W_self1, W_neigh1, b1.reshape(1, H), W_neigh2, W_self2)

    agg2 = _sc_aggregate(y2, src_p, dst_2d, with_deg=False)[0]

    out = pl.pallas_call(
        _tc_layer2_body,
        grid=(grid,),
        in_specs=[
            pl.BlockSpec((BR, C), lambda i: (i, 0)),
            pl.BlockSpec((NC, BR, C), lambda i: (0, i, 0)),
            pl.BlockSpec((BR, 8), lambda i: (i, 0)),
            pl.BlockSpec((1, C), lambda i: (0, 0)),
        ],
        out_specs=pl.BlockSpec((BR, C), lambda i: (i, 0)),
        out_shape=jax.ShapeDtypeStruct((N, C), jnp.float32),
    )(hs2, agg2, inv, b2.reshape(1, C))
    return out
